# baseline (speedup 1.0000x reference)
import jax
import jax.numpy as jnp
from jax import lax
from jax.experimental import pallas as pl
from jax.experimental.pallas import tpu as pltpu
from jax.experimental.pallas import tpu_sc as plsc

B, F, L, D, V = 1024, 26, 20, 128, 1000
N = B * F
NW = 32
PW = N // NW
CH = 4
ROWS = CH * L
NG = PW // CH
NBUF = 4
ZROW = F * (V + 1)
LANES = 16


def _take(vec, idx):
    dnums = lax.GatherDimensionNumbers(
        offset_dims=(), collapsed_slice_dims=(0,), start_index_map=(0,))
    return lax.gather(vec, idx[:, None], dnums, (1,),
                      mode=lax.GatherScatterMode.PROMISE_IN_BOUNDS)


def _body(ids_hbm, len_hbm, base_hbm, table_hbm, out_hbm,
          ids_v, len_v, base_v, scale_v, idx_v, rows_v, outc_v, gsems, osems):
    c = lax.axis_index("c")
    s = lax.axis_index("s")
    wid = s * 2 + c
    pstart = wid * PW

    pltpu.sync_copy(ids_hbm.at[pl.ds(pstart * L, PW * L)], ids_v)
    pltpu.sync_copy(len_hbm.at[pl.ds(pstart, PW)], len_v.at[pl.ds(0, PW)])
    pltpu.sync_copy(base_hbm.at[pl.ds(pstart, PW)], base_v.at[pl.ds(0, PW)])

    @pl.loop(0, PW // LANES)
    def _scale(k):
        l16 = len_v[pl.ds(k * LANES, LANES)]
        scale_v[pl.ds(k * LANES, LANES)] = 1.0 / jnp.maximum(l16, 1).astype(jnp.float32)

    lane = lax.iota(jnp.int32, LANES)
    segoff = []
    posoff = []
    for j in range(ROWS // LANES):
        q = lane + (j * LANES)
        so = lax.shift_right_logical(q * 205, 12)
        segoff.append(so)
        posoff.append(q - so * L)

    @pl.loop(0, NG)
    def _index(g):
        len16c = len_v[pl.ds(g * CH, LANES)]
        base16c = base_v[pl.ds(g * CH, LANES)]
        for j in range(ROWS // LANES):
            id16 = ids_v[pl.ds(g * ROWS + j * LANES, LANES)]
            len16 = _take(len16c, segoff[j])
            b16 = _take(base16c, segoff[j])
            idx = jnp.where(posoff[j] < len16, b16 + id16, ZROW)
            idx_v[g, pl.ds(j * LANES, LANES)] = idx

    for b in range(NBUF):
        pltpu.async_copy(table_hbm.at[idx_v.at[b]], rows_v.at[b], gsems[b])

    @pl.loop(0, NG, step=NBUF)
    def _pool(g0):
        for b in range(NBUF):
            g = g0 + b
            pltpu.make_async_copy(table_hbm.at[idx_v.at[g]],
                                  rows_v.at[b], gsems[b]).wait()

            @pl.when(g >= NBUF)
            def _():
                pltpu.make_async_copy(
                    outc_v.at[b],
                    out_hbm.at[pl.ds(pstart + (g - NBUF) * CH, CH)],
                    osems[b]).wait()

            scale16c = scale_v[pl.ds(g * CH, LANES)]
            for p in range(CH):
                sc = _take(scale16c, jnp.full((LANES,), p, jnp.int32))
                for cc in range(D // LANES):
                    acc = rows_v[b, p * L, pl.ds(cc * LANES, LANES)]
                    for l in range(1, L):
                        acc = acc + rows_v[b, p * L + l, pl.ds(cc * LANES, LANES)]
                    outc_v[b, p, pl.ds(cc * LANES, LANES)] = acc * sc
            pltpu.async_copy(outc_v.at[b],
                             out_hbm.at[pl.ds(pstart + g * CH, CH)], osems[b])

            @pl.when(g + NBUF < NG)
            def _():
                pltpu.async_copy(table_hbm.at[idx_v.at[g + NBUF]],
                                 rows_v.at[b], gsems[b])

    for b in range(NBUF):
        g = NG - NBUF + b
        pltpu.make_async_copy(outc_v.at[b],
                              out_hbm.at[pl.ds(pstart + g * CH, CH)],
                              osems[b]).wait()


@jax.jit
def _pooled(ids_flat, lens_flat, rowbase, table_aug):
    mesh = plsc.VectorSubcoreMesh(core_axis_name="c", subcore_axis_name="s")
    return pl.kernel(
        _body,
        out_type=jax.ShapeDtypeStruct((N, D), jnp.float32),
        mesh=mesh,
        scratch_types=[
            pltpu.VMEM((PW * L,), jnp.int32),
            pltpu.VMEM((PW + LANES,), jnp.int32),
            pltpu.VMEM((PW + LANES,), jnp.int32),
            pltpu.VMEM((PW + LANES,), jnp.float32),
            pltpu.VMEM((NG, ROWS), jnp.int32),
            pltpu.VMEM((NBUF, ROWS, D), jnp.float32),
            pltpu.VMEM((NBUF, CH, D), jnp.float32),
            [pltpu.SemaphoreType.DMA] * NBUF,
            [pltpu.SemaphoreType.DMA] * NBUF,
        ],
    )(ids_flat, lens_flat, rowbase, table_aug)


def kernel(dynamic_ids, dynamic_lengths, tables):
    ids_flat = dynamic_ids.astype(jnp.int32).reshape(N * L)
    lens_flat = dynamic_lengths.astype(jnp.int32).reshape(N)
    rowbase = jnp.tile(jnp.arange(F, dtype=jnp.int32) * (V + 1), B)
    table_aug = jnp.concatenate(
        [tables.reshape(F * (V + 1), D), jnp.zeros((1, D), jnp.float32)], axis=0)
    out = _pooled(ids_flat, lens_flat, rowbase, table_aug)
    return out.reshape(B, F, D)


# speedup vs baseline: 33.3136x; 33.3136x over previous
import jax
import jax.numpy as jnp
from jax import lax
from jax.experimental import pallas as pl
from jax.experimental.pallas import tpu as pltpu
from jax.experimental.pallas import tpu_sc as plsc

B, F, L, D, V = 1024, 26, 20, 128, 1000
N = B * F
NW = 32
PW = N // NW
CH = 4
ROWS = CH * L
NG = PW // CH
HS = 2
HROWS = HS * L
LANES = 16
FH = F // 2
ZROW = FH * (V + 1)
STG = 816
SROWS = 16 * STG


def _take(vec, idx):
    dnums = lax.GatherDimensionNumbers(
        offset_dims=(), collapsed_slice_dims=(0,), start_index_map=(0,))
    return lax.gather(vec, idx[:, None], dnums, (1,),
                      mode=lax.GatherScatterMode.PROMISE_IN_BOUNDS)


def _body(ids_hbm, len_hbm, table_hbm, out_hbm,
          len_v, idsb_v, idx_v, rows_v, outc_v, shared_v,
          isems, gsems, osems):
    c = lax.axis_index("c")
    s = lax.axis_index("s")
    wid = c * 16 + s
    pstart = wid * PW

    pltpu.sync_copy(table_hbm.at[c, pl.ds(s * STG, STG)],
                    shared_v.at[pl.ds(s * STG, STG)])
    plsc.subcore_barrier()

    pltpu.sync_copy(len_hbm.at[pl.ds(pstart, PW)], len_v.at[pl.ds(0, PW)])

    lane = lax.iota(jnp.int32, LANES)
    segoff = []
    posoff = []
    for j in range(ROWS // LANES):
        q = lane + (j * LANES)
        so = lax.shift_right_logical(q * 205, 12)
        segoff.append(so)
        posoff.append(q - so * L)

    def _fire_ids(g, b):
        pltpu.async_copy(ids_hbm.at[pl.ds((pstart + g * CH) * L, ROWS)],
                         idsb_v.at[b], isems[b])

    def _mkidx(g, b):
        len16c = len_v[pl.ds(g * CH, LANES)]
        for j in range(ROWS // LANES):
            id16 = idsb_v[b, pl.ds(j * LANES, LANES)]
            len16 = _take(len16c, segoff[j])
            n16 = (pstart + g * CH) + segoff[j]
            f16 = lax.shift_right_logical(n16, 10) - c * FH
            idx = jnp.where(posoff[j] < len16, f16 * (V + 1) + id16, ZROW)
            idx_v[b, pl.ds(j * LANES, LANES)] = idx

    _fire_ids(0, 0)
    pltpu.make_async_copy(ids_hbm.at[pl.ds(pstart * L, ROWS)],
                          idsb_v.at[0], isems[0]).wait()
    _mkidx(0, 0)
    _fire_ids(1, 1)
    for hl in range(2):
        pltpu.async_copy(
            shared_v.at[idx_v.at[0, pl.ds(hl * HROWS, HROWS)]],
            rows_v.at[hl], gsems[hl])

    def _fire_gather2(g, hl, par, rb):
        pltpu.async_copy(
            shared_v.at[idx_v.at[par, pl.ds(hl * HROWS, HROWS)]],
            rows_v.at[rb], gsems[rb])

    @pl.loop(0, NG, step=2)
    def _pool(g0):
        for par in range(2):
            g = g0 + par
            npar = (par + 1) % 2

            @pl.when(g + 1 < NG)
            def _():
                pltpu.make_async_copy(
                    ids_hbm.at[pl.ds((pstart + (g + 1) * CH) * L, ROWS)],
                    idsb_v.at[npar], isems[npar]).wait()
                _mkidx(g + 1, npar)

                @pl.when(g + 2 < NG)
                def _():
                    _fire_ids(g + 2, par)

            len16c = len_v[pl.ds(g * CH, LANES)]
            for hl in range(2):
                rb = hl
                pltpu.make_async_copy(
                    shared_v.at[idx_v.at[par, pl.ds(hl * HROWS, HROWS)]],
                    rows_v.at[rb], gsems[rb]).wait()

                if hl == 0:
                    @pl.when(g >= 2)
                    def _():
                        pltpu.make_async_copy(
                            outc_v.at[par],
                            out_hbm.at[pl.ds(pstart + (g - 2) * CH, CH)],
                            osems[par]).wait()

                for p in range(HS):
                    lb = _take(len16c, jnp.full((LANES,), hl * HS + p, jnp.int32))
                    sc = 1.0 / jnp.maximum(lb, 1).astype(jnp.float32)
                    for cc in range(D // LANES):
                        acc = rows_v[rb, p * L, pl.ds(cc * LANES, LANES)]
                        for l in range(1, L):
                            acc = acc + rows_v[rb, p * L + l, pl.ds(cc * LANES, LANES)]
                        outc_v[par, hl * HS + p, pl.ds(cc * LANES, LANES)] = acc * sc

                @pl.when(2 * g + hl + 2 < 2 * NG)
                def _():
                    _fire_gather2(g + 1, hl, npar, rb)

            pltpu.async_copy(outc_v.at[par],
                             out_hbm.at[pl.ds(pstart + g * CH, CH)],
                             osems[par])

    for b in range(2):
        g = NG - 2 + b
        pltpu.make_async_copy(outc_v.at[g % 2],
                              out_hbm.at[pl.ds(pstart + g * CH, CH)],
                              osems[g % 2]).wait()


@jax.jit
def _pooled(ids_flat, lens_flat, table_blk):
    mesh = plsc.VectorSubcoreMesh(core_axis_name="c", subcore_axis_name="s")
    return pl.kernel(
        _body,
        out_type=jax.ShapeDtypeStruct((N, D), jnp.float32),
        mesh=mesh,
        scratch_types=[
            pltpu.VMEM((PW + LANES,), jnp.int32),
            pltpu.VMEM((2, ROWS), jnp.int32),
            pltpu.VMEM((2, ROWS), jnp.int32),
            pltpu.VMEM((2, HROWS, D), jnp.float32),
            pltpu.VMEM((2, CH, D), jnp.float32),
            pltpu.VMEM_SHARED((SROWS, D), jnp.float32),
            [pltpu.SemaphoreType.DMA] * 2,
            [pltpu.SemaphoreType.DMA] * 2,
            [pltpu.SemaphoreType.DMA] * 2,
        ],
    )(ids_flat, lens_flat, table_blk)


def kernel(dynamic_ids, dynamic_lengths, tables):
    ids_flat = dynamic_ids.astype(jnp.int32).transpose(1, 0, 2).reshape(N * L)
    lens_flat = dynamic_lengths.astype(jnp.int32).T.reshape(N)
    blk = tables.reshape(2, FH * (V + 1), D)
    blk = jnp.pad(blk, ((0, 0), (0, SROWS - FH * (V + 1)), (0, 0)))
    out = _pooled(ids_flat, lens_flat, blk)
    return out.reshape(F, B, D).transpose(1, 0, 2)


# speedup vs baseline: 46.6249x; 1.3996x over previous
import jax
import jax.numpy as jnp
from jax import lax
from jax.experimental import pallas as pl
from jax.experimental.pallas import tpu as pltpu
from jax.experimental.pallas import tpu_sc as plsc

B, F, L, D, V = 1024, 26, 20, 128, 1000
N = B * F
NW = 32
PW = N // NW
CQ = 16
IDS = CQ * L
NG = PW // CQ
LANES = 16
FH = F // 2
ZROW = FH * (V + 1)
STG = 816
SROWS = 16 * STG


def _take(vec, idx):
    dnums = lax.GatherDimensionNumbers(
        offset_dims=(), collapsed_slice_dims=(0,), start_index_map=(0,))
    return lax.gather(vec, idx[:, None], dnums, (1,),
                      mode=lax.GatherScatterMode.PROMISE_IN_BOUNDS)


def _body(ids_hbm, len_hbm, table_hbm, out_hbm,
          len_v, idsb_v, idx_v, acc_v, outc_v, shared_v,
          isems, gsems, osems):
    c = lax.axis_index("c")
    s = lax.axis_index("s")
    wid = c * 16 + s
    pstart = wid * PW

    pltpu.sync_copy(table_hbm.at[c, pl.ds(s * STG, STG)],
                    shared_v.at[pl.ds(s * STG, STG)])
    plsc.subcore_barrier()

    pltpu.sync_copy(len_hbm.at[pl.ds(pstart, PW)], len_v.at[pl.ds(0, PW)])

    lane = lax.iota(jnp.int32, LANES)
    zero16 = jnp.zeros((LANES,), jnp.float32)

    for par in range(2):
        for p in range(CQ):
            for cc in range(D // LANES):
                acc_v[par, p, pl.ds(cc * LANES, LANES)] = zero16

    kbase = wid * NG

    def _fire_ids(g, b):
        pltpu.async_copy(ids_hbm.at[kbase + g], idsb_v.at[b], isems[b])

    def _mkidx(g, b):
        len16 = len_v[pl.ds(g * CQ, CQ)]
        n16 = (pstart + g * CQ) + lane
        base16 = (lax.shift_right_logical(n16, 10) - c * FH) * (V + 1)
        for l in range(L):
            id16 = idsb_v[b, l]
            idx = jnp.where(len16 > l, base16 + id16, ZROW)
            idx_v[b, l] = idx

    def _pool_copy(par, l):
        return pltpu.make_async_copy(
            shared_v.at[idx_v.at[par, l]], acc_v.at[par], gsems[par])

    def _fire_pool(par):
        for l in range(L):
            pltpu.async_copy(
                shared_v.at[idx_v.at[par, l]],
                acc_v.at[par], gsems[par], add=True)

    _fire_ids(0, 0)
    pltpu.make_async_copy(ids_hbm.at[kbase], idsb_v.at[0], isems[0]).wait()
    _mkidx(0, 0)
    _fire_ids(1, 1)
    _fire_pool(0)

    @pl.loop(0, NG, step=2)
    def _pool(g0):
        for par in range(2):
            g = g0 + par
            npar = (par + 1) % 2

            @pl.when(g + 1 < NG)
            def _():
                pltpu.make_async_copy(
                    ids_hbm.at[kbase + g + 1],
                    idsb_v.at[npar], isems[npar]).wait()
                _mkidx(g + 1, npar)
                _fire_pool(npar)

                @pl.when(g + 2 < NG)
                def _():
                    _fire_ids(g + 2, par)

            for l in range(L):
                _pool_copy(par, l).wait()

            @pl.when(g >= 2)
            def _():
                pltpu.make_async_copy(
                    outc_v.at[par],
                    out_hbm.at[pl.ds(pstart + (g - 2) * CQ, CQ)],
                    osems[par]).wait()

            len16c = len_v[pl.ds(g * CQ, CQ)]
            for p in range(CQ):
                lb = _take(len16c, jnp.full((LANES,), p, jnp.int32))
                sc = 1.0 / jnp.maximum(lb, 1).astype(jnp.float32)
                for cc in range(D // LANES):
                    v = acc_v[par, p, pl.ds(cc * LANES, LANES)]
                    outc_v[par, p, pl.ds(cc * LANES, LANES)] = v * sc
                    acc_v[par, p, pl.ds(cc * LANES, LANES)] = zero16

            pltpu.async_copy(outc_v.at[par],
                             out_hbm.at[pl.ds(pstart + g * CQ, CQ)],
                             osems[par])

    for b in range(2):
        g = NG - 2 + b
        pltpu.make_async_copy(outc_v.at[g % 2],
                              out_hbm.at[pl.ds(pstart + g * CQ, CQ)],
                              osems[g % 2]).wait()


@jax.jit
def _pooled(ids_t, lens_flat, table_blk):
    mesh = plsc.VectorSubcoreMesh(core_axis_name="c", subcore_axis_name="s")
    return pl.kernel(
        _body,
        out_type=jax.ShapeDtypeStruct((N, D), jnp.float32),
        mesh=mesh,
        scratch_types=[
            pltpu.VMEM((PW,), jnp.int32),
            pltpu.VMEM((2, L, CQ), jnp.int32),
            pltpu.VMEM((2, L, CQ), jnp.int32),
            pltpu.VMEM((2, CQ, D), jnp.float32),
            pltpu.VMEM((2, CQ, D), jnp.float32),
            pltpu.VMEM_SHARED((SROWS, D), jnp.float32),
            [pltpu.SemaphoreType.DMA] * 2,
            [pltpu.SemaphoreType.DMA] * 2,
            [pltpu.SemaphoreType.DMA] * 2,
        ],
    )(ids_t, lens_flat, table_blk)


def kernel(dynamic_ids, dynamic_lengths, tables):
    idsf = dynamic_ids.astype(jnp.int32).transpose(1, 0, 2).reshape(N, L)
    ids_t = idsf.reshape(N // CQ, CQ, L).transpose(0, 2, 1)
    lens_flat = dynamic_lengths.astype(jnp.int32).T.reshape(N)
    blk = tables.reshape(2, FH * (V + 1), D)
    blk = jnp.pad(blk, ((0, 0), (0, SROWS - FH * (V + 1)), (0, 0)))
    out = _pooled(ids_t, lens_flat, blk)
    return out.reshape(F, B, D).transpose(1, 0, 2)
